# baseline (speedup 1.0000x reference)
import functools

import jax
import jax.numpy as jnp
from jax import lax
from jax.experimental import pallas as pl
from jax.experimental.pallas import tpu as pltpu
from jax.experimental.pallas import tpu_sc as plsc

NUM_FACTORS = 64
BATCH = 16384
NC = 2
NS = 16
L = 16
NW = NC * NS
B_PER_W = BATCH // NW
D_CHUNKS = NUM_FACTORS // L


def _gmf_body(users_hbm, items_hbm, ut_hbm, it_hbm, w_hbm, b_hbm, out_hbm,
              idx_u, idx_i, u_rows, i_rows, w_v, b_v, out_v,
              sem_u, sem_i):
    wid = lax.axis_index("s") * NC + lax.axis_index("c")
    base = wid * B_PER_W

    pltpu.sync_copy(users_hbm.at[pl.ds(base, B_PER_W)], idx_u)
    pltpu.sync_copy(items_hbm.at[pl.ds(base, B_PER_W)], idx_i)
    pltpu.sync_copy(w_hbm, w_v)
    pltpu.sync_copy(b_hbm, b_v)

    cp_u = pltpu.async_copy(ut_hbm.at[idx_u], u_rows, sem_u)
    cp_i = pltpu.async_copy(it_hbm.at[idx_i], i_rows, sem_i)
    cp_u.wait()
    cp_i.wait()

    bvec = b_v[...]
    lanes = lax.iota(jnp.int32, L)

    w_chunks = [w_v[pl.ds(c * L, L)] for c in range(D_CHUNKS)]

    def group_body(g, _):
        rows = g * L + lanes
        acc = jnp.zeros((L,), jnp.float32)
        for c in range(D_CHUNKS):
            for j in range(L):
                d = c * L + j
                dv = jnp.full((L,), d, dtype=jnp.int32)
                uu = plsc.load_gather(u_rows, [rows, dv])
                vv = plsc.load_gather(i_rows, [rows, dv])
                acc = acc + (uu * vv) * w_chunks[c][j]
        x = acc + bvec
        out_v[pl.ds(g * L, L)] = 1.0 / (1.0 + jnp.exp(-x))
        return _

    lax.fori_loop(0, B_PER_W // L, group_body, None)

    pltpu.sync_copy(out_v, out_hbm.at[pl.ds(base, B_PER_W)])


@functools.cache
def _build_gmf_sc():
    return functools.partial(
        pl.kernel,
        mesh=plsc.VectorSubcoreMesh(core_axis_name="c", subcore_axis_name="s"),
        out_type=jax.ShapeDtypeStruct((BATCH,), jnp.float32),
        compiler_params=pltpu.CompilerParams(needs_layout_passes=False,
                                             use_tc_tiling_on_sc=False),
        scratch_types=[
            pltpu.VMEM((B_PER_W,), jnp.int32),
            pltpu.VMEM((B_PER_W,), jnp.int32),
            pltpu.VMEM((B_PER_W, NUM_FACTORS), jnp.float32),
            pltpu.VMEM((B_PER_W, NUM_FACTORS), jnp.float32),
            pltpu.VMEM((NUM_FACTORS,), jnp.float32),
            pltpu.VMEM((L,), jnp.float32),
            pltpu.VMEM((B_PER_W,), jnp.float32),
            pltpu.SemaphoreType.DMA,
            pltpu.SemaphoreType.DMA,
        ],
    )(_gmf_body)


def kernel(users, items, user_table, item_table, W, b):
    w_flat = W.reshape(NUM_FACTORS)
    b_vec = jnp.broadcast_to(b, (L,))
    return _build_gmf_sc()(users.astype(jnp.int32), items.astype(jnp.int32),
                           user_table, item_table, w_flat, b_vec)


# speedup vs baseline: 1.5134x; 1.5134x over previous
import functools

import jax
import jax.numpy as jnp
from jax import lax
from jax.experimental import pallas as pl
from jax.experimental.pallas import tpu as pltpu
from jax.experimental.pallas import tpu_sc as plsc

NUM_FACTORS = 64
BATCH = 16384
NC = 2
NS = 16
L = 16
NW = NC * NS
B_PER_W = BATCH // NW
D_CHUNKS = NUM_FACTORS // L
CH = 128
NCHUNK = B_PER_W // CH
K = 16


def _gmf_body(users_hbm, items_hbm, ut_hbm, it_hbm, w_hbm, b_hbm, out_hbm,
              su_v, si_v, u_rows, i_rows, w_v, b_v, out_v,
              sem_u, sem_i):
    wid = lax.axis_index("s") * NC + lax.axis_index("c")
    base = wid * B_PER_W

    pltpu.sync_copy(users_hbm.at[pl.ds(base, B_PER_W)], su_v)
    pltpu.sync_copy(items_hbm.at[pl.ds(base, B_PER_W)], si_v)
    pltpu.sync_copy(w_hbm, w_v)
    pltpu.sync_copy(b_hbm, b_v)

    w_chunks = [w_v[pl.ds(c * L, L)] for c in range(D_CHUNKS)]
    bvec = b_v[...]
    lanes = lax.iota(jnp.int32, L)

    def chunk_body(ch, _):
        cbase = ch * CH

        def fetch_body(s, _):
            rb = s * K
            uvec = su_v[pl.ds(cbase + rb, K)]
            ivec = si_v[pl.ds(cbase + rb, K)]
            cps = []
            for j in range(K):
                cps.append(pltpu.async_copy(ut_hbm.at[uvec[j]],
                                            u_rows.at[rb + j], sem_u))
                cps.append(pltpu.async_copy(it_hbm.at[ivec[j]],
                                            i_rows.at[rb + j], sem_i))
            for cp in cps:
                cp.wait()
            return _

        lax.fori_loop(0, CH // K, fetch_body, None)

        for g in range(CH // L):
            jv = g * L + lanes
            acc = jnp.zeros((L,), jnp.float32)
            for c in range(D_CHUNKS):
                for j in range(L):
                    d = c * L + j
                    dv = jnp.full((L,), d, dtype=jnp.int32)
                    uu = plsc.load_gather(u_rows, [jv, dv])
                    vv = plsc.load_gather(i_rows, [jv, dv])
                    acc = acc + (uu * vv) * w_chunks[c][j]
            x = acc + bvec
            out_v[pl.ds(cbase + g * L, L)] = 1.0 / (1.0 + jnp.exp(-x))
        return _

    lax.fori_loop(0, NCHUNK, chunk_body, None)

    pltpu.sync_copy(out_v, out_hbm.at[pl.ds(base, B_PER_W)])


@functools.cache
def _build_gmf_sc():
    return functools.partial(
        pl.kernel,
        mesh=plsc.VectorSubcoreMesh(core_axis_name="c", subcore_axis_name="s"),
        out_type=jax.ShapeDtypeStruct((BATCH,), jnp.float32),
        compiler_params=pltpu.CompilerParams(needs_layout_passes=False),
        scratch_types=[
            pltpu.VMEM((B_PER_W,), jnp.int32),
            pltpu.VMEM((B_PER_W,), jnp.int32),
            pltpu.VMEM((CH, NUM_FACTORS), jnp.float32),
            pltpu.VMEM((CH, NUM_FACTORS), jnp.float32),
            pltpu.VMEM((NUM_FACTORS,), jnp.float32),
            pltpu.VMEM((L,), jnp.float32),
            pltpu.VMEM((B_PER_W,), jnp.float32),
            pltpu.SemaphoreType.DMA,
            pltpu.SemaphoreType.DMA,
        ],
    )(_gmf_body)


def kernel(users, items, user_table, item_table, W, b):
    w_flat = W.reshape(NUM_FACTORS)
    b_vec = jnp.broadcast_to(b, (L,))
    return _build_gmf_sc()(users.astype(jnp.int32), items.astype(jnp.int32),
                           user_table, item_table, w_flat, b_vec)


# speedup vs baseline: 1.5136x; 1.0001x over previous
import functools

import jax
import jax.numpy as jnp
from jax import lax
from jax.experimental import pallas as pl
from jax.experimental.pallas import tpu as pltpu
from jax.experimental.pallas import tpu_sc as plsc

NUM_FACTORS = 64
BATCH = 16384
NC = 2
NS = 16
L = 16
NW = NC * NS
B_PER_W = BATCH // NW
D_CHUNKS = NUM_FACTORS // L
CH = 128
NCHUNK = B_PER_W // CH
K = 16


def _gmf_body(users_hbm, items_hbm, ut_hbm, it_hbm, w_hbm, b_hbm, out_hbm,
              su_v, si_v, u_rows, i_rows, w_v, b_v, out_v,
              sem_u, sem_i):
    wid = lax.axis_index("s") * NC + lax.axis_index("c")
    base = wid * B_PER_W

    pltpu.sync_copy(users_hbm.at[pl.ds(base, B_PER_W)], su_v)
    pltpu.sync_copy(items_hbm.at[pl.ds(base, B_PER_W)], si_v)
    pltpu.sync_copy(w_hbm, w_v)
    pltpu.sync_copy(b_hbm, b_v)

    w_chunks = [w_v[pl.ds(c * L, L)] for c in range(D_CHUNKS)]
    bvec = b_v[...]
    lanes = lax.iota(jnp.int32, L)

    def chunk_body(ch, _):
        cbase = ch * CH

        def fetch_body(s, _):
            rb = s * K
            uvec = su_v[pl.ds(cbase + rb, K)]
            ivec = si_v[pl.ds(cbase + rb, K)]
            cps = []
            for j in range(K):
                cps.append(pltpu.async_copy(ut_hbm.at[uvec[j]],
                                            u_rows.at[rb + j], sem_u))
                cps.append(pltpu.async_copy(it_hbm.at[ivec[j]],
                                            i_rows.at[rb + j], sem_i))
            for cp in cps:
                cp.wait()
            return _

        lax.fori_loop(0, CH // K, fetch_body, None)

        for g in range(CH // L):
            jv = g * L + lanes
            acc = jnp.zeros((L,), jnp.float32)
            for c in range(D_CHUNKS):
                for j in range(L):
                    d = c * L + j
                    dv = jnp.full((L,), d, dtype=jnp.int32)
                    uu = plsc.load_gather(u_rows, [jv, dv])
                    vv = plsc.load_gather(i_rows, [jv, dv])
                    acc = acc + (uu * vv) * w_chunks[c][j]
            x = acc + bvec
            out_v[pl.ds(cbase + g * L, L)] = 1.0 / (1.0 + jnp.exp(-x))
        return _

    lax.fori_loop(0, NCHUNK, chunk_body, None)

    pltpu.sync_copy(out_v, out_hbm.at[pl.ds(base, B_PER_W)])


@functools.cache
def _build_gmf_sc():
    return functools.partial(
        pl.kernel,
        mesh=plsc.VectorSubcoreMesh(core_axis_name="c", subcore_axis_name="s"),
        out_type=jax.ShapeDtypeStruct((BATCH,), jnp.float32),
        compiler_params=pltpu.CompilerParams(needs_layout_passes=False,
                                             use_tc_tiling_on_sc=True),
        scratch_types=[
            pltpu.VMEM((B_PER_W,), jnp.int32),
            pltpu.VMEM((B_PER_W,), jnp.int32),
            pltpu.VMEM((CH, NUM_FACTORS), jnp.float32),
            pltpu.VMEM((CH, NUM_FACTORS), jnp.float32),
            pltpu.VMEM((NUM_FACTORS,), jnp.float32),
            pltpu.VMEM((L,), jnp.float32),
            pltpu.VMEM((B_PER_W,), jnp.float32),
            pltpu.SemaphoreType.DMA,
            pltpu.SemaphoreType.DMA,
        ],
    )(_gmf_body)


def kernel(users, items, user_table, item_table, W, b):
    w_flat = W.reshape(NUM_FACTORS)
    b_vec = jnp.broadcast_to(b, (L,))
    return _build_gmf_sc()(users.astype(jnp.int32), items.astype(jnp.int32),
                           user_table, item_table, w_flat, b_vec)


# speedup vs baseline: 2.0810x; 1.3749x over previous
import functools

import jax
import jax.numpy as jnp
from jax import lax
from jax.experimental import pallas as pl
from jax.experimental.pallas import tpu as pltpu
from jax.experimental.pallas import tpu_sc as plsc

NUM_FACTORS = 64
BATCH = 16384
NC = 2
NS = 16
L = 16
NW = NC * NS
B_PER_W = BATCH // NW
D_CHUNKS = NUM_FACTORS // L
CH = 128
NCHUNK = B_PER_W // CH
G = 4
BLK = 128


def _gmf_body(users_hbm, items_hbm, ut_hbm, it_hbm, w_hbm, b_hbm, out_hbm,
              su_v, si_v,
              ub0, ub1, ub2, ub3, ib0, ib1, ib2, ib3,
              u_rows, i_rows, w_v, b_v, out_v,
              sem_u, sem_i):
    wid = lax.axis_index("s") * NC + lax.axis_index("c")
    base = wid * B_PER_W
    ubufs = [ub0, ub1, ub2, ub3]
    ibufs = [ib0, ib1, ib2, ib3]

    pltpu.sync_copy(users_hbm.at[pl.ds(base, B_PER_W)], su_v)
    pltpu.sync_copy(items_hbm.at[pl.ds(base, B_PER_W)], si_v)
    pltpu.sync_copy(w_hbm, w_v)
    pltpu.sync_copy(b_hbm, b_v)

    w_chunks = [w_v[pl.ds(c * L, L)] for c in range(D_CHUNKS)]
    bvec = b_v[...]
    lanes = lax.iota(jnp.int32, L)

    def chunk_body(ch, _):
        cbase = ch * CH

        def fetch_body(s, _):
            uvec = su_v[pl.ds(cbase + s * L, L)]
            ivec = si_v[pl.ds(cbase + s * L, L)]
            for q in range(L // G):
                cps = []
                rms = []
                for t in range(G):
                    r_u = uvec[q * G + t]
                    r_i = ivec[q * G + t]
                    off_u = pl.multiple_of((r_u // BLK) * BLK, BLK)
                    off_i = pl.multiple_of((r_i // BLK) * BLK, BLK)
                    rms.append((r_u % BLK, r_i % BLK))
                    cps.append(pltpu.async_copy(
                        ut_hbm.at[:, pl.ds(off_u, BLK)], ubufs[t], sem_u))
                    cps.append(pltpu.async_copy(
                        it_hbm.at[:, pl.ds(off_i, BLK)], ibufs[t], sem_i))
                for cp in cps:
                    cp.wait()
                for t in range(G):
                    j = cbase + s * L + q * G + t
                    rm_u = jnp.full((L,), rms[t][0], dtype=jnp.int32)
                    rm_i = jnp.full((L,), rms[t][1], dtype=jnp.int32)
                    for c in range(D_CHUNKS):
                        cl = c * L + lanes
                        u_rows[j - cbase, pl.ds(c * L, L)] = (
                            plsc.load_gather(ubufs[t], [cl, rm_u]))
                        i_rows[j - cbase, pl.ds(c * L, L)] = (
                            plsc.load_gather(ibufs[t], [cl, rm_i]))
            return _

        lax.fori_loop(0, CH // L, fetch_body, None)

        for g in range(CH // L):
            jv = g * L + lanes
            acc = jnp.zeros((L,), jnp.float32)
            for c in range(D_CHUNKS):
                for j in range(L):
                    d = c * L + j
                    dv = jnp.full((L,), d, dtype=jnp.int32)
                    uu = plsc.load_gather(u_rows, [jv, dv])
                    vv = plsc.load_gather(i_rows, [jv, dv])
                    acc = acc + (uu * vv) * w_chunks[c][j]
            x = acc + bvec
            out_v[pl.ds(cbase + g * L, L)] = 1.0 / (1.0 + jnp.exp(-x))
        return _

    lax.fori_loop(0, NCHUNK, chunk_body, None)

    pltpu.sync_copy(out_v, out_hbm.at[pl.ds(base, B_PER_W)])


@functools.cache
def _build_gmf_sc():
    return functools.partial(
        pl.kernel,
        mesh=plsc.VectorSubcoreMesh(core_axis_name="c", subcore_axis_name="s"),
        out_type=jax.ShapeDtypeStruct((BATCH,), jnp.float32),
        compiler_params=pltpu.CompilerParams(needs_layout_passes=False,
                                             use_tc_tiling_on_sc=True),
        scratch_types=(
            [pltpu.VMEM((B_PER_W,), jnp.int32)] * 2
            + [pltpu.VMEM((NUM_FACTORS, BLK), jnp.float32)] * 8
            + [pltpu.VMEM((CH, NUM_FACTORS), jnp.float32)] * 2
            + [pltpu.VMEM((NUM_FACTORS,), jnp.float32),
               pltpu.VMEM((L,), jnp.float32),
               pltpu.VMEM((B_PER_W,), jnp.float32),
               pltpu.SemaphoreType.DMA,
               pltpu.SemaphoreType.DMA]
        ),
    )(_gmf_body)


def kernel(users, items, user_table, item_table, W, b):
    w_flat = W.reshape(NUM_FACTORS)
    b_vec = jnp.broadcast_to(b, (L,))
    return _build_gmf_sc()(users.astype(jnp.int32), items.astype(jnp.int32),
                           user_table.T, item_table.T, w_flat, b_vec)


# speedup vs baseline: 2.3937x; 1.1502x over previous
import functools

import jax
import jax.numpy as jnp
from jax import lax
from jax.experimental import pallas as pl
from jax.experimental.pallas import tpu as pltpu
from jax.experimental.pallas import tpu_sc as plsc

NUM_FACTORS = 64
BATCH = 16384
NC = 2
NS = 16
L = 16
NW = NC * NS
B_PER_W = BATCH // NW
D_CHUNKS = NUM_FACTORS // L
CH = 128
NCHUNK = B_PER_W // CH
G = 2
BLK = 128


def _gmf_body(users_hbm, items_hbm, ut_hbm, it_hbm, w_hbm, b_hbm, out_hbm,
              su_v, si_v,
              ub0, ub1, ub2, ub3, ib0, ib1, ib2, ib3,
              u_rows, i_rows, w_v, b_v, out_v,
              sem_u, sem_i):
    wid = lax.axis_index("s") * NC + lax.axis_index("c")
    base = wid * B_PER_W
    ubufs = [ub0, ub1, ub2, ub3]
    ibufs = [ib0, ib1, ib2, ib3]

    pltpu.sync_copy(users_hbm.at[pl.ds(base, B_PER_W)], su_v)
    pltpu.sync_copy(items_hbm.at[pl.ds(base, B_PER_W)], si_v)
    pltpu.sync_copy(w_hbm, w_v)
    pltpu.sync_copy(b_hbm, b_v)

    w_chunks = [w_v[pl.ds(c * L, L)] for c in range(D_CHUNKS)]
    bvec = b_v[...]
    lanes = lax.iota(jnp.int32, L)

    def chunk_body(ch, _):
        cbase = ch * CH

        def fetch_body(s, _):
            uvec = su_v[pl.ds(cbase + s * L, L)]
            ivec = si_v[pl.ds(cbase + s * L, L)]
            nbatch = L // G

            def issue(q, slot):
                cps, rms = [], []
                for t in range(G):
                    r_u = uvec[q * G + t]
                    r_i = ivec[q * G + t]
                    off_u = pl.multiple_of((r_u // BLK) * BLK, BLK)
                    off_i = pl.multiple_of((r_i // BLK) * BLK, BLK)
                    rms.append((r_u % BLK, r_i % BLK))
                    cps.append(pltpu.async_copy(
                        ut_hbm.at[:, pl.ds(off_u, BLK)],
                        ubufs[slot * G + t], sem_u))
                    cps.append(pltpu.async_copy(
                        it_hbm.at[:, pl.ds(off_i, BLK)],
                        ibufs[slot * G + t], sem_i))
                return cps, rms

            def extract(q, slot, rms):
                for t in range(G):
                    j = s * L + q * G + t
                    rm_u = jnp.full((L,), rms[t][0], dtype=jnp.int32)
                    rm_i = jnp.full((L,), rms[t][1], dtype=jnp.int32)
                    for c in range(D_CHUNKS):
                        cl = c * L + lanes
                        u_rows[j, pl.ds(c * L, L)] = (
                            plsc.load_gather(ubufs[slot * G + t], [cl, rm_u]))
                        i_rows[j, pl.ds(c * L, L)] = (
                            plsc.load_gather(ibufs[slot * G + t], [cl, rm_i]))

            pending = issue(0, 0)
            for q in range(nbatch):
                nxt = issue(q + 1, (q + 1) % 2) if q + 1 < nbatch else None
                for cp in pending[0]:
                    cp.wait()
                extract(q, q % 2, pending[1])
                pending = nxt
            return _

        lax.fori_loop(0, CH // L, fetch_body, None)

        for g in range(CH // L):
            jv = g * L + lanes
            acc = jnp.zeros((L,), jnp.float32)
            for c in range(D_CHUNKS):
                for j in range(L):
                    d = c * L + j
                    dv = jnp.full((L,), d, dtype=jnp.int32)
                    uu = plsc.load_gather(u_rows, [jv, dv])
                    vv = plsc.load_gather(i_rows, [jv, dv])
                    acc = acc + (uu * vv) * w_chunks[c][j]
            x = acc + bvec
            out_v[pl.ds(cbase + g * L, L)] = 1.0 / (1.0 + jnp.exp(-x))
        return _

    lax.fori_loop(0, NCHUNK, chunk_body, None)

    pltpu.sync_copy(out_v, out_hbm.at[pl.ds(base, B_PER_W)])


@functools.cache
def _build_gmf_sc():
    return functools.partial(
        pl.kernel,
        mesh=plsc.VectorSubcoreMesh(core_axis_name="c", subcore_axis_name="s"),
        out_type=jax.ShapeDtypeStruct((BATCH,), jnp.float32),
        compiler_params=pltpu.CompilerParams(needs_layout_passes=False,
                                             use_tc_tiling_on_sc=True),
        scratch_types=(
            [pltpu.VMEM((B_PER_W,), jnp.int32)] * 2
            + [pltpu.VMEM((NUM_FACTORS, BLK), jnp.float32)] * 8
            + [pltpu.VMEM((CH, NUM_FACTORS), jnp.float32)] * 2
            + [pltpu.VMEM((NUM_FACTORS,), jnp.float32),
               pltpu.VMEM((L,), jnp.float32),
               pltpu.VMEM((B_PER_W,), jnp.float32),
               pltpu.SemaphoreType.DMA,
               pltpu.SemaphoreType.DMA]
        ),
    )(_gmf_body)


def kernel(users, items, user_table, item_table, W, b):
    w_flat = W.reshape(NUM_FACTORS)
    b_vec = jnp.broadcast_to(b, (L,))
    return _build_gmf_sc()(users.astype(jnp.int32), items.astype(jnp.int32),
                           user_table.T, item_table.T, w_flat, b_vec)


# speedup vs baseline: 2.4965x; 1.0430x over previous
import functools

import jax
import jax.numpy as jnp
from jax import lax
from jax.experimental import pallas as pl
from jax.experimental.pallas import tpu as pltpu
from jax.experimental.pallas import tpu_sc as plsc

NUM_FACTORS = 64
BATCH = 16384
NC = 2
NS = 16
L = 16
NW = NC * NS
B_PER_W = BATCH // NW
D_CHUNKS = NUM_FACTORS // L
CH = 128
NCHUNK = B_PER_W // CH
G = 2
BLK = 128


def _gmf_body(users_hbm, items_hbm, ut_hbm, it_hbm, w_hbm, b_hbm, out_hbm,
              su_v, si_v,
              ub0, ub1, ub2, ub3, ub4, ub5, ib0, ib1, ib2, ib3, ib4, ib5,
              u_rows, i_rows, w_v, b_v, out_v,
              sem_u, sem_i):
    wid = lax.axis_index("s") * NC + lax.axis_index("c")
    base = wid * B_PER_W
    ubufs = [ub0, ub1, ub2, ub3, ub4, ub5]
    ibufs = [ib0, ib1, ib2, ib3, ib4, ib5]

    pltpu.sync_copy(users_hbm.at[pl.ds(base, B_PER_W)], su_v)
    pltpu.sync_copy(items_hbm.at[pl.ds(base, B_PER_W)], si_v)
    pltpu.sync_copy(w_hbm, w_v)
    pltpu.sync_copy(b_hbm, b_v)

    w_chunks = [w_v[pl.ds(c * L, L)] for c in range(D_CHUNKS)]
    bvec = b_v[...]
    lanes = lax.iota(jnp.int32, L)

    def chunk_body(ch, _):
        cbase = ch * CH

        def fetch_body(s, _):
            uvec = su_v[pl.ds(cbase + s * L, L)]
            ivec = si_v[pl.ds(cbase + s * L, L)]
            nbatch = L // G

            def issue(q, slot):
                cps, rms = [], []
                for t in range(G):
                    r_u = uvec[q * G + t]
                    r_i = ivec[q * G + t]
                    off_u = pl.multiple_of((r_u // BLK) * BLK, BLK)
                    off_i = pl.multiple_of((r_i // BLK) * BLK, BLK)
                    rms.append((r_u % BLK, r_i % BLK))
                    cps.append(pltpu.async_copy(
                        ut_hbm.at[:, pl.ds(off_u, BLK)],
                        ubufs[slot * G + t], sem_u))
                    cps.append(pltpu.async_copy(
                        it_hbm.at[:, pl.ds(off_i, BLK)],
                        ibufs[slot * G + t], sem_i))
                return cps, rms

            def extract(q, slot, rms):
                for t in range(G):
                    j = s * L + q * G + t
                    rm_u = jnp.full((L,), rms[t][0], dtype=jnp.int32)
                    rm_i = jnp.full((L,), rms[t][1], dtype=jnp.int32)
                    for c in range(D_CHUNKS):
                        cl = c * L + lanes
                        u_rows[pl.ds(j * NUM_FACTORS + c * L, L)] = (
                            plsc.load_gather(ubufs[slot * G + t], [cl, rm_u]))
                        i_rows[pl.ds(j * NUM_FACTORS + c * L, L)] = (
                            plsc.load_gather(ibufs[slot * G + t], [cl, rm_i]))

            depth = 3
            pending = [issue(q, q) for q in range(depth - 1)]
            for q in range(nbatch):
                if q + depth - 1 < nbatch:
                    pending.append(issue(q + depth - 1, (q + depth - 1) % depth))
                cur = pending.pop(0)
                for cp in cur[0]:
                    cp.wait()
                extract(q, q % depth, cur[1])
            return _

        lax.fori_loop(0, CH // L, fetch_body, None)

        for g in range(CH // L):
            jv64 = (g * L + lanes) * NUM_FACTORS
            acc = jnp.zeros((L,), jnp.float32)
            for c in range(D_CHUNKS):
                for j in range(L):
                    d = c * L + j
                    uu = plsc.load_gather(u_rows, [jv64 + d])
                    vv = plsc.load_gather(i_rows, [jv64 + d])
                    acc = acc + (uu * vv) * w_chunks[c][j]
            x = acc + bvec
            out_v[pl.ds(cbase + g * L, L)] = 1.0 / (1.0 + jnp.exp(-x))
        return _

    lax.fori_loop(0, NCHUNK, chunk_body, None)

    pltpu.sync_copy(out_v, out_hbm.at[pl.ds(base, B_PER_W)])


@functools.cache
def _build_gmf_sc():
    return functools.partial(
        pl.kernel,
        mesh=plsc.VectorSubcoreMesh(core_axis_name="c", subcore_axis_name="s"),
        out_type=jax.ShapeDtypeStruct((BATCH,), jnp.float32),
        compiler_params=pltpu.CompilerParams(needs_layout_passes=False,
                                             use_tc_tiling_on_sc=True),
        scratch_types=(
            [pltpu.VMEM((B_PER_W,), jnp.int32)] * 2
            + [pltpu.VMEM((NUM_FACTORS, BLK), jnp.float32)] * 12
            + [pltpu.VMEM((CH * NUM_FACTORS,), jnp.float32)] * 2
            + [pltpu.VMEM((NUM_FACTORS,), jnp.float32),
               pltpu.VMEM((L,), jnp.float32),
               pltpu.VMEM((B_PER_W,), jnp.float32),
               pltpu.SemaphoreType.DMA,
               pltpu.SemaphoreType.DMA]
        ),
    )(_gmf_body)


def kernel(users, items, user_table, item_table, W, b):
    w_flat = W.reshape(NUM_FACTORS)
    b_vec = jnp.broadcast_to(b, (L,))
    return _build_gmf_sc()(users.astype(jnp.int32), items.astype(jnp.int32),
                           user_table.T, item_table.T, w_flat, b_vec)


# speedup vs baseline: 2.6210x; 1.0498x over previous
import functools

import jax
import jax.numpy as jnp
from jax import lax
from jax.experimental import pallas as pl
from jax.experimental.pallas import tpu as pltpu
from jax.experimental.pallas import tpu_sc as plsc

NUM_FACTORS = 64
BATCH = 16384
NC = 2
NS = 16
L = 16
NW = NC * NS
B_PER_W = BATCH // NW
D_CHUNKS = NUM_FACTORS // L
CH = 128
NCHUNK = B_PER_W // CH
G = 1
BLK = 128


def _gmf_body(users_hbm, items_hbm, ut_hbm, it_hbm, w_hbm, b_hbm, out_hbm,
              su_v, si_v,
              ub0, ub1, ub2, ub3, ub4, ub5, ib0, ib1, ib2, ib3, ib4, ib5,
              u_rows, i_rows, w_v, b_v, out_v,
              sem_u, sem_i):
    wid = lax.axis_index("s") * NC + lax.axis_index("c")
    base = wid * B_PER_W
    ubufs = [ub0, ub1, ub2, ub3, ub4, ub5]
    ibufs = [ib0, ib1, ib2, ib3, ib4, ib5]

    pltpu.sync_copy(users_hbm.at[pl.ds(base, B_PER_W)], su_v)
    pltpu.sync_copy(items_hbm.at[pl.ds(base, B_PER_W)], si_v)
    pltpu.sync_copy(w_hbm, w_v)
    pltpu.sync_copy(b_hbm, b_v)

    w_chunks = [w_v[pl.ds(c * L, L)] for c in range(D_CHUNKS)]
    bvec = b_v[...]
    lanes = lax.iota(jnp.int32, L)

    def chunk_body(ch, _):
        cbase = ch * CH

        def fetch_body(s, _):
            uvec = su_v[pl.ds(cbase + s * L, L)]
            ivec = si_v[pl.ds(cbase + s * L, L)]
            nbatch = L // G

            def issue(q, slot):
                cps, rms = [], []
                for t in range(G):
                    r_u = uvec[q * G + t]
                    r_i = ivec[q * G + t]
                    off_u = pl.multiple_of((r_u // BLK) * BLK, BLK)
                    off_i = pl.multiple_of((r_i // BLK) * BLK, BLK)
                    rms.append((r_u % BLK, r_i % BLK))
                    cps.append(pltpu.async_copy(
                        ut_hbm.at[:, pl.ds(off_u, BLK)],
                        ubufs[slot * G + t], sem_u))
                    cps.append(pltpu.async_copy(
                        it_hbm.at[:, pl.ds(off_i, BLK)],
                        ibufs[slot * G + t], sem_i))
                return cps, rms

            def extract(q, slot, rms):
                for t in range(G):
                    j = s * L + q * G + t
                    rm_u = jnp.full((L,), rms[t][0], dtype=jnp.int32)
                    rm_i = jnp.full((L,), rms[t][1], dtype=jnp.int32)
                    for c in range(D_CHUNKS):
                        cl = c * L + lanes
                        u_rows[pl.ds(j * NUM_FACTORS + c * L, L)] = (
                            plsc.load_gather(ubufs[slot * G + t], [cl, rm_u]))
                        i_rows[pl.ds(j * NUM_FACTORS + c * L, L)] = (
                            plsc.load_gather(ibufs[slot * G + t], [cl, rm_i]))

            depth = 6
            pending = [issue(q, q) for q in range(depth - 1)]
            for q in range(nbatch):
                if q + depth - 1 < nbatch:
                    pending.append(issue(q + depth - 1, (q + depth - 1) % depth))
                cur = pending.pop(0)
                for cp in cur[0]:
                    cp.wait()
                extract(q, q % depth, cur[1])
            return _

        lax.fori_loop(0, CH // L, fetch_body, None)

        for g in range(CH // L):
            jv64 = (g * L + lanes) * NUM_FACTORS
            acc = jnp.zeros((L,), jnp.float32)
            for c in range(D_CHUNKS):
                for j in range(L):
                    d = c * L + j
                    uu = plsc.load_gather(u_rows, [jv64 + d])
                    vv = plsc.load_gather(i_rows, [jv64 + d])
                    acc = acc + (uu * vv) * w_chunks[c][j]
            x = acc + bvec
            out_v[pl.ds(cbase + g * L, L)] = 1.0 / (1.0 + jnp.exp(-x))
        return _

    lax.fori_loop(0, NCHUNK, chunk_body, None)

    pltpu.sync_copy(out_v, out_hbm.at[pl.ds(base, B_PER_W)])


@functools.cache
def _build_gmf_sc():
    return functools.partial(
        pl.kernel,
        mesh=plsc.VectorSubcoreMesh(core_axis_name="c", subcore_axis_name="s"),
        out_type=jax.ShapeDtypeStruct((BATCH,), jnp.float32),
        compiler_params=pltpu.CompilerParams(needs_layout_passes=False,
                                             use_tc_tiling_on_sc=True),
        scratch_types=(
            [pltpu.VMEM((B_PER_W,), jnp.int32)] * 2
            + [pltpu.VMEM((NUM_FACTORS, BLK), jnp.float32)] * 12
            + [pltpu.VMEM((CH * NUM_FACTORS,), jnp.float32)] * 2
            + [pltpu.VMEM((NUM_FACTORS,), jnp.float32),
               pltpu.VMEM((L,), jnp.float32),
               pltpu.VMEM((B_PER_W,), jnp.float32),
               pltpu.SemaphoreType.DMA,
               pltpu.SemaphoreType.DMA]
        ),
    )(_gmf_body)


def kernel(users, items, user_table, item_table, W, b):
    w_flat = W.reshape(NUM_FACTORS)
    b_vec = jnp.broadcast_to(b, (L,))
    return _build_gmf_sc()(users.astype(jnp.int32), items.astype(jnp.int32),
                           user_table.T, item_table.T, w_flat, b_vec)


# speedup vs baseline: 2.6336x; 1.0048x over previous
import functools

import jax
import jax.numpy as jnp
from jax import lax
from jax.experimental import pallas as pl
from jax.experimental.pallas import tpu as pltpu
from jax.experimental.pallas import tpu_sc as plsc

NUM_FACTORS = 64
BATCH = 16384
NC = 2
NS = 16
L = 16
NW = NC * NS
B_PER_W = BATCH // NW
D_CHUNKS = NUM_FACTORS // L
CH = 64
NCHUNK = B_PER_W // CH
G = 1
BLK = 128


def _gmf_body(users_hbm, items_hbm, ut_hbm, it_hbm, w_hbm, b_hbm, out_hbm,
              su_v, si_v,
              ub0, ub1, ub2, ub3, ub4, ub5, ub6,
              ib0, ib1, ib2, ib3, ib4, ib5, ib6,
              u_rows, i_rows, w_v, b_v, out_v,
              sem_u, sem_i):
    wid = lax.axis_index("s") * NC + lax.axis_index("c")
    base = wid * B_PER_W
    ubufs = [ub0, ub1, ub2, ub3, ub4, ub5, ub6]
    ibufs = [ib0, ib1, ib2, ib3, ib4, ib5, ib6]

    pltpu.sync_copy(users_hbm.at[pl.ds(base, B_PER_W)], su_v)
    pltpu.sync_copy(items_hbm.at[pl.ds(base, B_PER_W)], si_v)
    pltpu.sync_copy(w_hbm, w_v)
    pltpu.sync_copy(b_hbm, b_v)

    w_chunks = [w_v[pl.ds(c * L, L)] for c in range(D_CHUNKS)]
    bvec = b_v[...]
    lanes = lax.iota(jnp.int32, L)

    def chunk_body(ch, _):
        cbase = ch * CH

        def fetch_body(s, _):
            uvec = su_v[pl.ds(cbase + s * L, L)]
            ivec = si_v[pl.ds(cbase + s * L, L)]
            nbatch = L // G

            def issue(q, slot):
                cps, rms = [], []
                for t in range(G):
                    r_u = uvec[q * G + t]
                    r_i = ivec[q * G + t]
                    off_u = pl.multiple_of((r_u // BLK) * BLK, BLK)
                    off_i = pl.multiple_of((r_i // BLK) * BLK, BLK)
                    rms.append((r_u % BLK, r_i % BLK))
                    cps.append(pltpu.async_copy(
                        ut_hbm.at[:, pl.ds(off_u, BLK)],
                        ubufs[slot * G + t], sem_u))
                    cps.append(pltpu.async_copy(
                        it_hbm.at[:, pl.ds(off_i, BLK)],
                        ibufs[slot * G + t], sem_i))
                return cps, rms

            def extract(q, slot, rms):
                for t in range(G):
                    j = s * L + q * G + t
                    rm_u = jnp.full((L,), rms[t][0], dtype=jnp.int32)
                    rm_i = jnp.full((L,), rms[t][1], dtype=jnp.int32)
                    for c in range(D_CHUNKS):
                        cl = c * L + lanes
                        u_rows[pl.ds(j * NUM_FACTORS + c * L, L)] = (
                            plsc.load_gather(ubufs[slot * G + t], [cl, rm_u]))
                        i_rows[pl.ds(j * NUM_FACTORS + c * L, L)] = (
                            plsc.load_gather(ibufs[slot * G + t], [cl, rm_i]))

            depth = 7
            pending = [issue(q, q) for q in range(depth - 1)]
            for q in range(nbatch):
                if q + depth - 1 < nbatch:
                    pending.append(issue(q + depth - 1, (q + depth - 1) % depth))
                cur = pending.pop(0)
                for cp in cur[0]:
                    cp.wait()
                extract(q, q % depth, cur[1])
            return _

        lax.fori_loop(0, CH // L, fetch_body, None)

        for g in range(CH // L):
            jv64 = (g * L + lanes) * NUM_FACTORS
            acc = jnp.zeros((L,), jnp.float32)
            for c in range(D_CHUNKS):
                for j in range(L):
                    d = c * L + j
                    uu = plsc.load_gather(u_rows, [jv64 + d])
                    vv = plsc.load_gather(i_rows, [jv64 + d])
                    acc = acc + (uu * vv) * w_chunks[c][j]
            x = acc + bvec
            out_v[pl.ds(cbase + g * L, L)] = 1.0 / (1.0 + jnp.exp(-x))
        return _

    lax.fori_loop(0, NCHUNK, chunk_body, None)

    pltpu.sync_copy(out_v, out_hbm.at[pl.ds(base, B_PER_W)])


@functools.cache
def _build_gmf_sc():
    return functools.partial(
        pl.kernel,
        mesh=plsc.VectorSubcoreMesh(core_axis_name="c", subcore_axis_name="s"),
        out_type=jax.ShapeDtypeStruct((BATCH,), jnp.float32),
        compiler_params=pltpu.CompilerParams(needs_layout_passes=False,
                                             use_tc_tiling_on_sc=True),
        scratch_types=(
            [pltpu.VMEM((B_PER_W,), jnp.int32)] * 2
            + [pltpu.VMEM((NUM_FACTORS, BLK), jnp.float32)] * 14
            + [pltpu.VMEM((CH * NUM_FACTORS,), jnp.float32)] * 2
            + [pltpu.VMEM((NUM_FACTORS,), jnp.float32),
               pltpu.VMEM((L,), jnp.float32),
               pltpu.VMEM((B_PER_W,), jnp.float32),
               pltpu.SemaphoreType.DMA,
               pltpu.SemaphoreType.DMA]
        ),
    )(_gmf_body)


def kernel(users, items, user_table, item_table, W, b):
    w_flat = W.reshape(NUM_FACTORS)
    b_vec = jnp.broadcast_to(b, (L,))
    return _build_gmf_sc()(users.astype(jnp.int32), items.astype(jnp.int32),
                           user_table.T, item_table.T, w_flat, b_vec)


# speedup vs baseline: 2.6557x; 1.0084x over previous
import functools

import jax
import jax.numpy as jnp
from jax import lax
from jax.experimental import pallas as pl
from jax.experimental.pallas import tpu as pltpu
from jax.experimental.pallas import tpu_sc as plsc

NUM_FACTORS = 64
BATCH = 16384
NC = 2
NS = 16
L = 16
NW = NC * NS
B_PER_W = BATCH // NW
D_CHUNKS = NUM_FACTORS // L
CH = 256
NCHUNK = B_PER_W // CH
RING = 4
BLK = 128


def _gmf_body(users_hbm, items_hbm, ut_hbm, it_hbm, w_hbm, b_hbm, out_hbm,
              su_v, si_v, ub0, ub1, ub2, ub3, ib0, ib1, ib2, ib3,
              u_rows, i_rows, w_v, b_v, out_v, sem_u, sem_i):
    wid = lax.axis_index("s") * NC + lax.axis_index("c")
    base = wid * B_PER_W
    ubufs = [ub0, ub1, ub2, ub3]
    ibufs = [ib0, ib1, ib2, ib3]

    pltpu.sync_copy(users_hbm.at[pl.ds(base, B_PER_W)], su_v)
    pltpu.sync_copy(items_hbm.at[pl.ds(base, B_PER_W)], si_v)
    pltpu.sync_copy(w_hbm, w_v)
    pltpu.sync_copy(b_hbm, b_v)

    w_chunks = [w_v[pl.ds(c * L, L)] for c in range(D_CHUNKS)]
    bvec = b_v[...]
    lanes = lax.iota(jnp.int32, L)

    def issue_row(r_u, r_i, slot):
        off_u = pl.multiple_of((r_u // BLK) * BLK, BLK)
        off_i = pl.multiple_of((r_i // BLK) * BLK, BLK)
        pltpu.async_copy(ut_hbm.at[:, pl.ds(off_u, BLK)], ubufs[slot], sem_u)
        pltpu.async_copy(it_hbm.at[:, pl.ds(off_i, BLK)], ibufs[slot], sem_i)

    def drain_extract(r_u, r_i, slot, jdst):
        pltpu.make_async_copy(ut_hbm.at[:, pl.ds(0, BLK)],
                              ubufs[slot], sem_u).wait()
        pltpu.make_async_copy(it_hbm.at[:, pl.ds(0, BLK)],
                              ibufs[slot], sem_i).wait()
        rm_u = jnp.full((L,), r_u % BLK, dtype=jnp.int32)
        rm_i = jnp.full((L,), r_i % BLK, dtype=jnp.int32)
        for c in range(D_CHUNKS):
            cl = c * L + lanes
            u_rows[pl.ds(jdst * NUM_FACTORS + c * L, L)] = (
                plsc.load_gather(ubufs[slot], [cl, rm_u]))
            i_rows[pl.ds(jdst * NUM_FACTORS + c * L, L)] = (
                plsc.load_gather(ibufs[slot], [cl, rm_i]))

    def chunk_body(ch, _):
        cbase = ch * CH

        def win_body(s, carry):
            puvec, pivec = carry
            uvec = su_v[pl.ds(cbase + s * L, L)]
            ivec = si_v[pl.ds(cbase + s * L, L)]
            for j in range(L):
                slot = j % RING
                if j < RING:
                    @pl.when(s > 0)
                    def _drain():
                        drain_extract(puvec[L - RING + j], pivec[L - RING + j],
                                      slot, s * L + j - RING)
                else:
                    drain_extract(uvec[j - RING], ivec[j - RING],
                                  slot, s * L + j - RING)
                issue_row(uvec[j], ivec[j], slot)
            return uvec, ivec

        luvec, livec = lax.fori_loop(
            0, CH // L, win_body,
            (jnp.zeros((L,), jnp.int32), jnp.zeros((L,), jnp.int32)))
        for j in range(L - RING, L):
            drain_extract(luvec[j], livec[j], j % RING,
                          CH - RING + (j - (L - RING)))

        def red_body(g, _):
            jv64 = (g * L + lanes) * NUM_FACTORS
            acc = jnp.zeros((L,), jnp.float32)
            for c in range(D_CHUNKS):
                for j in range(L):
                    d = c * L + j
                    uu = plsc.load_gather(u_rows, [jv64 + d])
                    vv = plsc.load_gather(i_rows, [jv64 + d])
                    acc = acc + (uu * vv) * w_chunks[c][j]
            x = acc + bvec
            out_v[pl.ds(cbase + g * L, L)] = 1.0 / (1.0 + jnp.exp(-x))
            return _

        lax.fori_loop(0, CH // L, red_body, None)
        return _

    lax.fori_loop(0, NCHUNK, chunk_body, None)

    pltpu.sync_copy(out_v, out_hbm.at[pl.ds(base, B_PER_W)])


@functools.cache
def _build_gmf_sc():
    return functools.partial(
        pl.kernel,
        mesh=plsc.VectorSubcoreMesh(core_axis_name="c", subcore_axis_name="s"),
        out_type=jax.ShapeDtypeStruct((BATCH,), jnp.float32),
        compiler_params=pltpu.CompilerParams(needs_layout_passes=False,
                                             use_tc_tiling_on_sc=True),
        scratch_types=(
            [pltpu.VMEM((B_PER_W,), jnp.int32)] * 2
            + [pltpu.VMEM((NUM_FACTORS, BLK), jnp.float32)] * (2 * RING)
            + [pltpu.VMEM((CH * NUM_FACTORS,), jnp.float32)] * 2
            + [pltpu.VMEM((NUM_FACTORS,), jnp.float32),
               pltpu.VMEM((L,), jnp.float32),
               pltpu.VMEM((B_PER_W,), jnp.float32),
               pltpu.SemaphoreType.DMA,
               pltpu.SemaphoreType.DMA]
        ),
    )(_gmf_body)


def kernel(users, items, user_table, item_table, W, b):
    w_flat = W.reshape(NUM_FACTORS)
    b_vec = jnp.broadcast_to(b, (L,))
    return _build_gmf_sc()(users.astype(jnp.int32), items.astype(jnp.int32),
                           user_table.T, item_table.T, w_flat, b_vec)


# speedup vs baseline: 2.6585x; 1.0010x over previous
import functools

import jax
import jax.numpy as jnp
from jax import lax
from jax.experimental import pallas as pl
from jax.experimental.pallas import tpu as pltpu
from jax.experimental.pallas import tpu_sc as plsc

NUM_FACTORS = 64
BATCH = 16384
NC = 2
NS = 16
L = 16
NW = NC * NS
B_PER_W = BATCH // NW
D_CHUNKS = NUM_FACTORS // L
CH = 256
NCHUNK = B_PER_W // CH
RING = 4
BLK = 128


def _gmf_body(users_hbm, items_hbm, ut_hbm, it_hbm, w_hbm, b_hbm, out_hbm,
              su_v, si_v, ub0, ub1, ub2, ub3, ib0, ib1, ib2, ib3,
              u_rows, i_rows, w_v, b_v, out_v, sem_u, sem_i):
    wid = lax.axis_index("s") * NC + lax.axis_index("c")
    base = wid * B_PER_W
    ubufs = [ub0, ub1, ub2, ub3]
    ibufs = [ib0, ib1, ib2, ib3]

    pltpu.sync_copy(users_hbm.at[pl.ds(base, B_PER_W)], su_v)
    pltpu.sync_copy(items_hbm.at[pl.ds(base, B_PER_W)], si_v)
    pltpu.sync_copy(w_hbm, w_v)
    pltpu.sync_copy(b_hbm, b_v)

    w_chunks = [w_v[pl.ds(c * L, L)] for c in range(D_CHUNKS)]
    bvec = b_v[...]
    lanes = lax.iota(jnp.int32, L)

    def issue_row(r_u, r_i, slot):
        off_u = pl.multiple_of((r_u // BLK) * BLK, BLK)
        off_i = pl.multiple_of((r_i // BLK) * BLK, BLK)
        pltpu.async_copy(ut_hbm.at[:, pl.ds(off_u, BLK)], ubufs[slot], sem_u)
        pltpu.async_copy(it_hbm.at[:, pl.ds(off_i, BLK)], ibufs[slot], sem_i)

    def drain_extract(r_u, r_i, slot, jdst):
        rm_u = jnp.full((L,), r_u % BLK, dtype=jnp.int32)
        rm_i = jnp.full((L,), r_i % BLK, dtype=jnp.int32)
        pltpu.make_async_copy(ut_hbm.at[:, pl.ds(0, BLK)],
                              ubufs[slot], sem_u).wait()
        for c in range(D_CHUNKS):
            cl = c * L + lanes
            u_rows[pl.ds(jdst * NUM_FACTORS + c * L, L)] = (
                plsc.load_gather(ubufs[slot], [cl, rm_u]))
        pltpu.make_async_copy(it_hbm.at[:, pl.ds(0, BLK)],
                              ibufs[slot], sem_i).wait()
        for c in range(D_CHUNKS):
            cl = c * L + lanes
            i_rows[pl.ds(jdst * NUM_FACTORS + c * L, L)] = (
                plsc.load_gather(ibufs[slot], [cl, rm_i]))

    def chunk_body(ch, _):
        cbase = ch * CH

        def win_body(s, carry):
            puvec, pivec = carry
            uvec = su_v[pl.ds(cbase + s * L, L)]
            ivec = si_v[pl.ds(cbase + s * L, L)]
            for j in range(L):
                slot = j % RING
                if j < RING:
                    @pl.when(s > 0)
                    def _drain():
                        drain_extract(puvec[L - RING + j], pivec[L - RING + j],
                                      slot, s * L + j - RING)
                else:
                    drain_extract(uvec[j - RING], ivec[j - RING],
                                  slot, s * L + j - RING)
                issue_row(uvec[j], ivec[j], slot)
            return uvec, ivec

        luvec, livec = lax.fori_loop(
            0, CH // L, win_body,
            (jnp.zeros((L,), jnp.int32), jnp.zeros((L,), jnp.int32)))
        for j in range(L - RING, L):
            drain_extract(luvec[j], livec[j], j % RING,
                          CH - RING + (j - (L - RING)))

        def red_body(g, _):
            jv64 = (g * L + lanes) * NUM_FACTORS
            acc = jnp.zeros((L,), jnp.float32)
            for c in range(D_CHUNKS):
                for j in range(L):
                    d = c * L + j
                    uu = plsc.load_gather(u_rows, [jv64 + d])
                    vv = plsc.load_gather(i_rows, [jv64 + d])
                    acc = acc + (uu * vv) * w_chunks[c][j]
            x = acc + bvec
            out_v[pl.ds(cbase + g * L, L)] = 1.0 / (1.0 + jnp.exp(-x))
            return _

        lax.fori_loop(0, CH // L, red_body, None)
        return _

    lax.fori_loop(0, NCHUNK, chunk_body, None)

    pltpu.sync_copy(out_v, out_hbm.at[pl.ds(base, B_PER_W)])


@functools.cache
def _build_gmf_sc():
    return functools.partial(
        pl.kernel,
        mesh=plsc.VectorSubcoreMesh(core_axis_name="c", subcore_axis_name="s"),
        out_type=jax.ShapeDtypeStruct((BATCH,), jnp.float32),
        compiler_params=pltpu.CompilerParams(needs_layout_passes=False,
                                             use_tc_tiling_on_sc=True),
        scratch_types=(
            [pltpu.VMEM((B_PER_W,), jnp.int32)] * 2
            + [pltpu.VMEM((NUM_FACTORS, BLK), jnp.float32)] * (2 * RING)
            + [pltpu.VMEM((CH * NUM_FACTORS,), jnp.float32)] * 2
            + [pltpu.VMEM((NUM_FACTORS,), jnp.float32),
               pltpu.VMEM((L,), jnp.float32),
               pltpu.VMEM((B_PER_W,), jnp.float32),
               pltpu.SemaphoreType.DMA,
               pltpu.SemaphoreType.DMA]
        ),
    )(_gmf_body)


def kernel(users, items, user_table, item_table, W, b):
    w_flat = W.reshape(NUM_FACTORS)
    b_vec = jnp.broadcast_to(b, (L,))
    return _build_gmf_sc()(users.astype(jnp.int32), items.astype(jnp.int32),
                           user_table.T, item_table.T, w_flat, b_vec)
